# baseline (speedup 1.0000x reference)
import functools

import jax
import jax.numpy as jnp
from jax import lax
from jax.experimental import pallas as pl
from jax.experimental.pallas import tpu as pltpu
from jax.experimental.pallas import tpu_sc as plsc

CUTOFF = 5.0
NBASIS = 20


def _mlp_body(s_ref, w1_ref, b1_ref, w2_ref, b2_ref, o_ref):
    x = s_ref[...]
    h = jnp.dot(x, w1_ref[...], preferred_element_type=jnp.float32,
                precision=lax.Precision.HIGHEST) + b1_ref[...]
    h = h * jax.nn.sigmoid(h)
    o_ref[...] = jnp.dot(h, w2_ref[...], preferred_element_type=jnp.float32,
                         precision=lax.Precision.HIGHEST) + b2_ref[...]


def _lo16(x):
    return lax.bitcast_convert_type(lax.shift_left(x, 16), jnp.float32)


def _hi16(x):
    return lax.bitcast_convert_type(
        lax.bitwise_and(x, jnp.int32(-65536)), jnp.float32)


def _payload_body(u_ref, d4_ref, g_ref, w4_ref,
                  pa_ref, p0_ref, p1_ref, p2_ref):
    d4 = d4_ref[...]
    uc = d4 * (1.0 / CUTOFF)
    uc2 = uc * uc
    uc3 = uc2 * uc
    uc6 = uc3 * uc3
    uc7 = uc6 * uc
    uc8 = uc7 * uc
    env = 1.0 - 28.0 * uc6 + 48.0 * uc7 - 21.0 * uc8
    env = jnp.where(d4 < CUTOFF, env, 0.0)

    B4 = d4.shape[0]
    nl = lax.broadcasted_iota(jnp.int32, (B4, 128), 1)
    nb = (nl % 32 + 1).astype(jnp.float32)
    sin4 = jnp.sin(nb * (jnp.pi / CUTOFF) * d4)
    besf = sin4 * (jnp.sqrt(2.0 / CUTOFF) / d4 * env)
    besf = jnp.where(nl % 32 == 31, env, besf)
    filt4 = jnp.dot(besf, w4_ref[...],
                    preferred_element_type=jnp.float32)

    pk = g_ref[...]
    u = u_ref[...]
    TD = pk.shape[1] // 4
    D = TD // 3
    pas, p0s, p1s, p2s = [], [], [], []
    for e in range(4):
        base = e * TD
        a = _lo16(pk[:, base:base + D])
        vz = _hi16(pk[:, base:base + D])
        b = _lo16(pk[:, base + D:base + 2 * D])
        c = _hi16(pk[:, base + D:base + 2 * D])
        vx = _lo16(pk[:, base + 2 * D:base + 3 * D])
        vy = _hi16(pk[:, base + 2 * D:base + 3 * D])
        bb = b * filt4[:, base + D:base + 2 * D]
        cc = c * filt4[:, base + 2 * D:base + 3 * D]
        pas.append(a * filt4[:, base:base + D])
        p0s.append(bb * u[:, 3 * e:3 * e + 1] + cc * vx)
        p1s.append(bb * u[:, 3 * e + 1:3 * e + 2] + cc * vy)
        p2s.append(bb * u[:, 3 * e + 2:3 * e + 3] + cc * vz)
    pa_ref[...] = jnp.concatenate(pas, axis=1)
    p0_ref[...] = jnp.concatenate(p0s, axis=1)
    p1_ref[...] = jnp.concatenate(p1s, axis=1)
    p2_ref[...] = jnp.concatenate(p2s, axis=1)


def _mlp(scalar_embeddings, w1, b1, w2, b2):
    N, D = scalar_embeddings.shape
    TD = w2.shape[1]
    BN = 2000
    grid = (N // BN,)
    return pl.pallas_call(
        _mlp_body,
        grid=grid,
        in_specs=[
            pl.BlockSpec((BN, D), lambda i: (i, 0)),
            pl.BlockSpec((D, D), lambda i: (0, 0)),
            pl.BlockSpec((1, D), lambda i: (0, 0)),
            pl.BlockSpec((D, TD), lambda i: (0, 0)),
            pl.BlockSpec((1, TD), lambda i: (0, 0)),
        ],
        out_specs=pl.BlockSpec((BN, TD), lambda i: (i, 0)),
        out_shape=jax.ShapeDtypeStruct((N, TD), jnp.float32),
    )(scalar_embeddings, w1, b1.reshape(1, D), w2, b2.reshape(1, TD))


def _payloads(u3, d4, g_all, w4):
    E, TD = g_all.shape
    D = TD // 3
    E4 = E // 4
    BE4 = 400
    grid = (E4 // BE4,)
    u4 = u3.reshape(E4, 12)
    g4 = g_all.reshape(E4, 4 * TD)
    outs = pl.pallas_call(
        _payload_body,
        grid=grid,
        in_specs=[
            pl.BlockSpec((BE4, 12), lambda i: (i, 0)),
            pl.BlockSpec((BE4, 128), lambda i: (i, 0)),
            pl.BlockSpec((BE4, 4 * TD), lambda i: (i, 0)),
            pl.BlockSpec((128, 4 * TD), lambda i: (0, 0)),
        ],
        out_specs=[pl.BlockSpec((BE4, 4 * D), lambda i: (i, 0))] * 4,
        out_shape=[jax.ShapeDtypeStruct((E4, 4 * D), jnp.float32)] * 4,
    )(u4, d4, g4, w4)
    return tuple(o.reshape(E, D) for o in outs)


BLK = 128


def _sc_gather(t_all, neigh_pad):
    N, TD = t_all.shape
    E = neigh_pad.shape[0] - BLK
    nblk = E // BLK
    NW = 32
    base_cnt = nblk // NW
    n_extra = nblk - NW * base_cnt
    cnt_max = base_cnt + (1 if n_extra else 0)
    pairs = (cnt_max + 1) // 2
    mesh = plsc.VectorSubcoreMesh(core_axis_name="c", subcore_axis_name="s")

    @functools.partial(
        pl.kernel,
        out_type=jax.ShapeDtypeStruct((E, TD), jnp.int32),
        mesh=mesh,
        scratch_types=[
            pltpu.VMEM((cnt_max * BLK,), jnp.int32),
            pltpu.VMEM((BLK, TD), jnp.int32),
            pltpu.VMEM((BLK, TD), jnp.int32),
            pltpu.SemaphoreType.DMA,
            pltpu.SemaphoreType.DMA,
            pltpu.SemaphoreType.DMA,
            pltpu.SemaphoreType.DMA,
        ],
    )
    def k(t_hbm, idx_hbm, g_hbm, idx_all, buf0, buf1, gs0, gs1, ws0, ws1):
        wid = lax.axis_index("s") * 2 + lax.axis_index("c")
        base_blk = wid * base_cnt + jnp.minimum(wid, n_extra)
        cnt = jnp.where(wid < n_extra, base_cnt + 1, base_cnt)
        ebase = base_blk * BLK

        pltpu.sync_copy(idx_hbm.at[pl.ds(ebase, cnt_max * BLK)], idx_all)

        bufs = (buf0, buf1)
        gsems = (gs0, gs1)
        wsems = (ws0, ws1)

        def gissue(i, b):
            pltpu.async_copy(
                t_hbm.at[idx_all.at[pl.ds(i * BLK, BLK)]], bufs[b], gsems[b])

        def gwait(b):
            pltpu.make_async_copy(
                t_hbm.at[pl.ds(0, BLK)], bufs[b], gsems[b]).wait()

        def wissue(i, b):
            pltpu.async_copy(
                bufs[b], g_hbm.at[pl.ds(ebase + i * BLK, BLK)], wsems[b])

        def wwait(b):
            pltpu.make_async_copy(
                bufs[b], g_hbm.at[pl.ds(0, BLK)], wsems[b]).wait()

        gissue(0, 0)

        @pl.loop(0, pairs)
        def _(ii):
            for b in range(2):
                i = ii * 2 + b

                @pl.when(i < cnt)
                def _(i=i, b=b):
                    gwait(b)
                    wissue(i, b)

                    @pl.when(i + 1 < cnt)
                    def _():
                        @pl.when(i >= 1)
                        def _():
                            wwait(1 - b)

                        gissue(i + 1, 1 - b)

        wwait(0)
        wwait(1)

    return k(t_all, neigh_pad)


def _sc_scatter(central2d, pa, p0, p1, p2, zeros_nd):
    E, D = pa.shape
    N = zeros_nd.shape[0]
    nblk = pa.shape[0] // BLK
    NS = 16
    CH = ((nblk + NS - 1) // NS + 7) // 8 * 8
    n_full = nblk // CH
    rem = nblk - n_full * CH
    cnt_max = CH
    NBUF = 2
    rounds = (cnt_max + NBUF - 1) // NBUF
    tiles = N // 8
    rows8 = (tiles // NS) * 8
    n_hi = tiles - NS * (tiles // NS)
    assert n_hi * (rows8 + 8) + (NS - n_hi) * rows8 == N
    mesh = plsc.VectorSubcoreMesh(core_axis_name="c", subcore_axis_name="s")

    @functools.partial(
        pl.kernel,
        out_type=(jax.ShapeDtypeStruct((N, D), jnp.float32),
                  jax.ShapeDtypeStruct((N, D), jnp.float32),
                  jax.ShapeDtypeStruct((N, D), jnp.float32),
                  jax.ShapeDtypeStruct((N, D), jnp.float32)),
        mesh=mesh,
        scratch_types=[
            pltpu.VMEM((cnt_max, BLK), jnp.int32),
            pltpu.VMEM((BLK, D), jnp.float32),
            pltpu.VMEM((BLK, D), jnp.float32),
            pltpu.VMEM_SHARED((N, D), jnp.float32),
            pltpu.SemaphoreType.DMA,
            pltpu.SemaphoreType.DMA,
            pltpu.SemaphoreType.DMA,
            pltpu.SemaphoreType.DMA,
        ],
    )
    def k(c_hbm, pa_hbm, p0_hbm, p1_hbm, p2_hbm, z_hbm,
          ds_hbm, v0_hbm, v1_hbm, v2_hbm,
          idx2, b0, b1,
          acc, ps0, ps1, ss0, ss1):
        cid = lax.axis_index("c")
        sid = lax.axis_index("s")
        sbase = sid * CH
        cnt = jnp.where(sid < n_full, CH, rem)
        bufs = (b0, b1)
        psems = (ps0, ps1)
        ssems = (ss0, ss1)

        pltpu.sync_copy(c_hbm.at[pl.ds(sbase, CH)], idx2)

        def chunk(pay_hbm, out_hbm):
            @pl.when(sid == 0)
            def _():
                pltpu.sync_copy(z_hbm, acc)

            plsc.subcore_barrier()

            def pissue(i, b):
                pltpu.async_copy(
                    pay_hbm.at[pl.ds((sbase + i) * BLK, BLK)],
                    bufs[b], psems[b])

            def pwait(b):
                pltpu.make_async_copy(
                    pay_hbm.at[pl.ds(0, BLK)], bufs[b], psems[b]).wait()

            def sissue(i, b):
                pltpu.async_copy(bufs[b], acc.at[idx2.at[i]], ssems[b],
                                 add=True)

            def swait(b):
                pltpu.make_async_copy(
                    pay_hbm.at[pl.ds(0, BLK)], bufs[b], ssems[b]).wait()

            for b in range(NBUF - 1):
                @pl.when(b < cnt)
                def _(b=b):
                    pissue(b, b)

            @pl.loop(0, rounds)
            def _(ii):
                for b in range(NBUF):
                    i = ii * NBUF + b

                    @pl.when(i < cnt)
                    def _(i=i, b=b):
                        pwait(b)
                        sissue(i, b)
                        j = i + NBUF - 1

                        @pl.when(j < cnt)
                        def _():
                            nb = (b + NBUF - 1) % NBUF

                            @pl.when(j >= NBUF)
                            def _():
                                swait(nb)

                            pissue(j, nb)

            for b in range(NBUF):
                swait(b)

            plsc.subcore_barrier()

            @pl.when(sid < n_hi)
            def _():
                start = sid * (rows8 + 8)
                pltpu.sync_copy(acc.at[pl.ds(start, rows8 + 8)],
                                out_hbm.at[pl.ds(start, rows8 + 8)])

            @pl.when(sid >= n_hi)
            def _():
                start = n_hi * (rows8 + 8) + (sid - n_hi) * rows8
                pltpu.sync_copy(acc.at[pl.ds(start, rows8)],
                                out_hbm.at[pl.ds(start, rows8)])

            plsc.subcore_barrier()

        @pl.when(cid == 0)
        def _():
            chunk(pa_hbm, ds_hbm)
            chunk(p0_hbm, v0_hbm)

        @pl.when(cid == 1)
        def _():
            chunk(p1_hbm, v1_hbm)
            chunk(p2_hbm, v2_hbm)

    return k(central2d, pa, p0, p1, p2, zeros_nd)


def _pack2(x, y):
    s = jnp.stack([x.astype(jnp.bfloat16), y.astype(jnp.bfloat16)], axis=-1)
    return lax.bitcast_convert_type(s, jnp.int32)


def kernel(vector_embeddings, scalar_embeddings, edge_vectors, edge_index,
           Wf, bf, w1, b1, w2, b2):
    N, D = scalar_embeddings.shape
    TD = 3 * D
    E = edge_vectors.shape[0]
    central = edge_index[0].astype(jnp.int32)
    neigh = edge_index[1].astype(jnp.int32)

    nf = _mlp(scalar_embeddings, w1, b1, w2, b2)
    t_all = jnp.concatenate([
        _pack2(nf[:, :D], vector_embeddings[:, :, 2]),
        _pack2(nf[:, D:2 * D], nf[:, 2 * D:]),
        _pack2(vector_embeddings[:, :, 0], vector_embeddings[:, :, 1]),
    ], axis=1)

    neigh_pad = jnp.concatenate([neigh, jnp.zeros((BLK,), jnp.int32)])
    g_all = _sc_gather(t_all, neigh_pad)

    d = jnp.sqrt(jnp.sum(edge_vectors * edge_vectors, axis=-1, keepdims=True))
    u3 = edge_vectors / d
    d4 = jnp.repeat(d.reshape(E // 4, 4), 32, axis=1)
    wf32 = jnp.zeros((32, TD), jnp.float32).at[:NBASIS].set(Wf).at[31].set(bf)
    w4 = jax.scipy.linalg.block_diag(wf32, wf32, wf32, wf32)
    pa, p0, p1, p2 = _payloads(u3, d4, g_all, w4)

    zeros_nd = jnp.zeros((N, D), jnp.float32)
    nblk = E // BLK
    ch = ((nblk + 15) // 16 + 7) // 8 * 8
    central2d = jnp.concatenate([
        central.reshape(nblk, BLK),
        jnp.zeros((16 * ch - nblk, BLK), jnp.int32)])
    ds, v0, v1, v2 = _sc_scatter(central2d, pa, p0, p1, p2, zeros_nd)

    delta_v = jnp.stack([v0, v1, v2], axis=-1)
    return (delta_v, ds)


# speedup vs baseline: 1.5593x; 1.5593x over previous
import functools

import jax
import jax.numpy as jnp
from jax import lax
from jax.experimental import pallas as pl
from jax.experimental.pallas import tpu as pltpu
from jax.experimental.pallas import tpu_sc as plsc

CUTOFF = 5.0
NBASIS = 20


def _mlp_body(s_ref, w1_ref, b1_ref, w2_ref, b2_ref, o_ref):
    x = s_ref[...]
    h = jnp.dot(x, w1_ref[...], preferred_element_type=jnp.float32) + b1_ref[...]
    h = h * jax.nn.sigmoid(h)
    o_ref[...] = jnp.dot(h, w2_ref[...],
                         preferred_element_type=jnp.float32) + b2_ref[...]


def _lo16(x):
    return lax.bitcast_convert_type(lax.shift_left(x, 16), jnp.float32)


def _hi16(x):
    return lax.bitcast_convert_type(
        lax.bitwise_and(x, jnp.int32(-65536)), jnp.float32)


def _payload_body(u_ref, d4_ref, g_ref, w4_ref,
                  pa_ref, p0_ref, p1_ref, p2_ref):
    d4 = d4_ref[...]
    uc = d4 * (1.0 / CUTOFF)
    uc2 = uc * uc
    uc3 = uc2 * uc
    uc6 = uc3 * uc3
    uc7 = uc6 * uc
    uc8 = uc7 * uc
    env = 1.0 - 28.0 * uc6 + 48.0 * uc7 - 21.0 * uc8
    env = jnp.where(d4 < CUTOFF, env, 0.0)

    SUB = d4.shape[0]
    nl = lax.broadcasted_iota(jnp.int32, (SUB, 128), 1)
    nb = (nl % 32 + 1).astype(jnp.float32)
    sin4 = jnp.sin(nb * (jnp.pi / CUTOFF) * d4)
    besf = sin4 * (jnp.sqrt(2.0 / CUTOFF) / d4 * env)
    besf = jnp.where(nl % 32 == 31, env, besf)
    filt4 = jnp.dot(besf, w4_ref[...],
                    preferred_element_type=jnp.float32)

    TD = filt4.shape[1] // 4
    D = TD // 3
    for e in range(4):
        rs = pl.ds(e * SUB, SUB)
        base = e * TD
        a = _lo16(g_ref[rs, :D])
        vz = _hi16(g_ref[rs, :D])
        b = _lo16(g_ref[rs, D:2 * D])
        c = _hi16(g_ref[rs, D:2 * D])
        vx = _lo16(g_ref[rs, 2 * D:])
        vy = _hi16(g_ref[rs, 2 * D:])
        bb = b * filt4[:, base + D:base + 2 * D]
        cc = c * filt4[:, base + 2 * D:base + 3 * D]
        pa_ref[rs, :] = a * filt4[:, base:base + D]
        p0_ref[rs, :] = bb * u_ref[rs, 0:1] + cc * vx
        p1_ref[rs, :] = bb * u_ref[rs, 1:2] + cc * vy
        p2_ref[rs, :] = bb * u_ref[rs, 2:3] + cc * vz


def _mlp(scalar_embeddings, w1, b1, w2, b2):
    N, D = scalar_embeddings.shape
    TD = w2.shape[1]
    BN = 2000
    grid = (N // BN,)
    return pl.pallas_call(
        _mlp_body,
        grid=grid,
        in_specs=[
            pl.BlockSpec((BN, D), lambda i: (i, 0)),
            pl.BlockSpec((D, D), lambda i: (0, 0)),
            pl.BlockSpec((1, D), lambda i: (0, 0)),
            pl.BlockSpec((D, TD), lambda i: (0, 0)),
            pl.BlockSpec((1, TD), lambda i: (0, 0)),
        ],
        out_specs=pl.BlockSpec((BN, TD), lambda i: (i, 0)),
        out_shape=jax.ShapeDtypeStruct((N, TD), jnp.float32),
    )(scalar_embeddings, w1, b1.reshape(1, D), w2, b2.reshape(1, TD))


BE = 1600


def _payloads(u3, d4, g_all, w4):
    E, TD = g_all.shape
    D = TD // 3
    grid = (E // BE,)
    return pl.pallas_call(
        _payload_body,
        grid=grid,
        in_specs=[
            pl.BlockSpec((BE, 3), lambda i: (i, 0)),
            pl.BlockSpec((BE // 4, 128), lambda i: (i, 0)),
            pl.BlockSpec((BE, TD), lambda i: (i, 0)),
            pl.BlockSpec((128, 4 * TD), lambda i: (0, 0)),
        ],
        out_specs=[pl.BlockSpec((BE, D), lambda i: (i, 0))] * 4,
        out_shape=[jax.ShapeDtypeStruct((E, D), jnp.float32)] * 4,
    )(u3, d4, g_all, w4)


BLK = 128


def _sc_gather(t_all, neigh_pad):
    N, TD = t_all.shape
    E = neigh_pad.shape[0] - BLK
    nblk = E // BLK
    NW = 32
    base_cnt = nblk // NW
    n_extra = nblk - NW * base_cnt
    cnt_max = base_cnt + (1 if n_extra else 0)
    pairs = (cnt_max + 1) // 2
    mesh = plsc.VectorSubcoreMesh(core_axis_name="c", subcore_axis_name="s")

    @functools.partial(
        pl.kernel,
        out_type=jax.ShapeDtypeStruct((E, TD), jnp.int32),
        mesh=mesh,
        scratch_types=[
            pltpu.VMEM((cnt_max * BLK,), jnp.int32),
            pltpu.VMEM((BLK, TD), jnp.int32),
            pltpu.VMEM((BLK, TD), jnp.int32),
            pltpu.SemaphoreType.DMA,
            pltpu.SemaphoreType.DMA,
            pltpu.SemaphoreType.DMA,
            pltpu.SemaphoreType.DMA,
        ],
    )
    def k(t_hbm, idx_hbm, g_hbm, idx_all, buf0, buf1, gs0, gs1, ws0, ws1):
        wid = lax.axis_index("s") * 2 + lax.axis_index("c")
        base_blk = wid * base_cnt + jnp.minimum(wid, n_extra)
        cnt = jnp.where(wid < n_extra, base_cnt + 1, base_cnt)
        ebase = base_blk * BLK

        pltpu.sync_copy(idx_hbm.at[pl.ds(ebase, cnt_max * BLK)], idx_all)

        bufs = (buf0, buf1)
        gsems = (gs0, gs1)
        wsems = (ws0, ws1)

        def gissue(i, b):
            pltpu.async_copy(
                t_hbm.at[idx_all.at[pl.ds(i * BLK, BLK)]], bufs[b], gsems[b])

        def gwait(b):
            pltpu.make_async_copy(
                t_hbm.at[pl.ds(0, BLK)], bufs[b], gsems[b]).wait()

        def wissue(i, b):
            pltpu.async_copy(
                bufs[b], g_hbm.at[pl.ds(ebase + i * BLK, BLK)], wsems[b])

        def wwait(b):
            pltpu.make_async_copy(
                bufs[b], g_hbm.at[pl.ds(0, BLK)], wsems[b]).wait()

        gissue(0, 0)

        @pl.loop(0, pairs)
        def _(ii):
            for b in range(2):
                i = ii * 2 + b

                @pl.when(i < cnt)
                def _(i=i, b=b):
                    gwait(b)
                    wissue(i, b)

                    @pl.when(i + 1 < cnt)
                    def _():
                        @pl.when(i >= 1)
                        def _():
                            wwait(1 - b)

                        gissue(i + 1, 1 - b)

        wwait(0)
        wwait(1)

    return k(t_all, neigh_pad)


def _sc_scatter(c2d_1, c2d_2, pa1, pa2, p01, p02, p11, p12, p21, p22,
                zeros_nd):
    EH, D = pa1.shape
    N = zeros_nd.shape[0]
    nblk = EH // BLK
    NS = 16
    NSH = 8
    CH = ((nblk + NSH - 1) // NSH + 7) // 8 * 8
    n_full = nblk // CH
    rem = nblk - n_full * CH
    cnt_max = CH
    NBUF = 2
    rounds = (cnt_max + NBUF - 1) // NBUF
    tiles = N // 8
    rows8 = (tiles // NS) * 8
    n_hi = tiles - NS * (tiles // NS)
    assert n_hi * (rows8 + 8) + (NS - n_hi) * rows8 == N
    mesh = plsc.VectorSubcoreMesh(core_axis_name="c", subcore_axis_name="s")

    @functools.partial(
        pl.kernel,
        out_type=(jax.ShapeDtypeStruct((N, D), jnp.float32),
                  jax.ShapeDtypeStruct((N, D), jnp.float32),
                  jax.ShapeDtypeStruct((N, D), jnp.float32),
                  jax.ShapeDtypeStruct((N, D), jnp.float32)),
        mesh=mesh,
        scratch_types=[
            pltpu.VMEM((cnt_max, BLK), jnp.int32),
            pltpu.VMEM((BLK, D), jnp.float32),
            pltpu.VMEM((BLK, D), jnp.float32),
            pltpu.VMEM_SHARED((N, D), jnp.float32),
            pltpu.SemaphoreType.DMA,
            pltpu.SemaphoreType.DMA,
            pltpu.SemaphoreType.DMA,
            pltpu.SemaphoreType.DMA,
        ],
    )
    def k(c1_hbm, c2_hbm, pa1_hbm, pa2_hbm, p01_hbm, p02_hbm,
          p11_hbm, p12_hbm, p21_hbm, p22_hbm, z_hbm,
          ds_hbm, v0_hbm, v1_hbm, v2_hbm,
          idx2, b0, b1,
          acc, ps0, ps1, ss0, ss1):
        cid = lax.axis_index("c")
        sid = lax.axis_index("s")
        sidh = lax.rem(sid, NSH)
        sbase = sidh * CH
        cnt = jnp.where(sidh < n_full, CH, rem)
        bufs = (b0, b1)
        psems = (ps0, ps1)
        ssems = (ss0, ss1)

        @pl.when(sid < NSH)
        def _():
            pltpu.sync_copy(c1_hbm.at[pl.ds(sbase, CH)], idx2)

        @pl.when(sid >= NSH)
        def _():
            pltpu.sync_copy(c2_hbm.at[pl.ds(sbase, CH)], idx2)

        def pipe(pay_hbm):
            def pissue(i, b):
                pltpu.async_copy(
                    pay_hbm.at[pl.ds((sbase + i) * BLK, BLK)],
                    bufs[b], psems[b])

            def pwait(b):
                pltpu.make_async_copy(
                    pay_hbm.at[pl.ds(0, BLK)], bufs[b], psems[b]).wait()

            def sissue(i, b):
                pltpu.async_copy(bufs[b], acc.at[idx2.at[i]], ssems[b],
                                 add=True)

            def swait(b):
                pltpu.make_async_copy(
                    pay_hbm.at[pl.ds(0, BLK)], bufs[b], ssems[b]).wait()

            for b in range(NBUF - 1):
                @pl.when(b < cnt)
                def _(b=b):
                    pissue(b, b)

            @pl.loop(0, rounds)
            def _(ii):
                for b in range(NBUF):
                    i = ii * NBUF + b

                    @pl.when(i < cnt)
                    def _(i=i, b=b):
                        pwait(b)
                        sissue(i, b)
                        j = i + NBUF - 1

                        @pl.when(j < cnt)
                        def _():
                            nb = (b + NBUF - 1) % NBUF

                            @pl.when(j >= NBUF)
                            def _():
                                swait(nb)

                            pissue(j, nb)

            for b in range(NBUF):
                swait(b)

        def chunk(payh1_hbm, payh2_hbm, out_hbm):
            @pl.when(sid == 0)
            def _():
                pltpu.sync_copy(z_hbm, acc)

            plsc.subcore_barrier()

            @pl.when(sid < NSH)
            def _():
                pipe(payh1_hbm)

            @pl.when(sid >= NSH)
            def _():
                pipe(payh2_hbm)

            plsc.subcore_barrier()

            @pl.when(sid < n_hi)
            def _():
                start = sid * (rows8 + 8)
                pltpu.sync_copy(acc.at[pl.ds(start, rows8 + 8)],
                                out_hbm.at[pl.ds(start, rows8 + 8)])

            @pl.when(sid >= n_hi)
            def _():
                start = n_hi * (rows8 + 8) + (sid - n_hi) * rows8
                pltpu.sync_copy(acc.at[pl.ds(start, rows8)],
                                out_hbm.at[pl.ds(start, rows8)])

            plsc.subcore_barrier()

        @pl.when(cid == 0)
        def _():
            chunk(pa1_hbm, pa2_hbm, ds_hbm)
            chunk(p01_hbm, p02_hbm, v0_hbm)

        @pl.when(cid == 1)
        def _():
            chunk(p11_hbm, p12_hbm, v1_hbm)
            chunk(p21_hbm, p22_hbm, v2_hbm)

    return k(c2d_1, c2d_2, pa1, pa2, p01, p02, p11, p12, p21, p22, zeros_nd)


def _pack2(x, y):
    s = jnp.stack([x.astype(jnp.bfloat16), y.astype(jnp.bfloat16)], axis=-1)
    return lax.bitcast_convert_type(s, jnp.int32)


def kernel(vector_embeddings, scalar_embeddings, edge_vectors, edge_index,
           Wf, bf, w1, b1, w2, b2):
    N, D = scalar_embeddings.shape
    TD = 3 * D
    E = edge_vectors.shape[0]
    central = edge_index[0].astype(jnp.int32)
    neigh = edge_index[1].astype(jnp.int32)

    nf = _mlp(scalar_embeddings, w1, b1, w2, b2)
    t_all = jnp.concatenate([
        _pack2(nf[:, :D], vector_embeddings[:, :, 2]),
        _pack2(nf[:, D:2 * D], nf[:, 2 * D:]),
        _pack2(vector_embeddings[:, :, 0], vector_embeddings[:, :, 1]),
    ], axis=1)

    d = jnp.sqrt(jnp.sum(edge_vectors * edge_vectors, axis=-1, keepdims=True))
    u3 = edge_vectors / d
    d4 = d.reshape(E // BE, 4, BE // 4).transpose(0, 2, 1).reshape(E // 4, 4)
    d4 = jnp.repeat(d4, 32, axis=1)
    wf32 = jnp.zeros((32, TD), jnp.float32).at[:NBASIS].set(Wf).at[31].set(bf)
    w4 = jax.scipy.linalg.block_diag(wf32, wf32, wf32, wf32)

    EH = E // 2
    pays = []
    for h in range(2):
        sl = slice(h * EH, (h + 1) * EH)
        neigh_pad = jnp.concatenate([neigh[sl], jnp.zeros((BLK,), jnp.int32)])
        g_h = _sc_gather(t_all, neigh_pad)
        pays.append(_payloads(u3[sl], d4[h * (EH // 4):(h + 1) * (EH // 4)],
                              g_h, w4))

    zeros_nd = jnp.zeros((N, D), jnp.float32)
    nblk_h = EH // BLK
    ch = ((nblk_h + 7) // 8 + 7) // 8 * 8
    c2d = []
    for h in range(2):
        c_h = central[h * EH:(h + 1) * EH].reshape(nblk_h, BLK)
        c2d.append(jnp.concatenate(
            [c_h, jnp.zeros((8 * ch - nblk_h, BLK), jnp.int32)]))
    (pa1, p01, p11, p21), (pa2, p02, p12, p22) = pays
    ds, v0, v1, v2 = _sc_scatter(c2d[0], c2d[1], pa1, pa2, p01, p02,
                                 p11, p12, p21, p22, zeros_nd)

    delta_v = jnp.stack([v0, v1, v2], axis=-1)
    return (delta_v, ds)


# speedup vs baseline: 1.5784x; 1.0123x over previous
import functools

import jax
import jax.numpy as jnp
from jax import lax
from jax.experimental import pallas as pl
from jax.experimental.pallas import tpu as pltpu
from jax.experimental.pallas import tpu_sc as plsc

CUTOFF = 5.0
NBASIS = 20


def _pack_pair(x, y):
    xi = lax.convert_element_type(
        lax.bitcast_convert_type(
            lax.convert_element_type(x, jnp.bfloat16), jnp.int16), jnp.int32)
    yi = lax.convert_element_type(
        lax.bitcast_convert_type(
            lax.convert_element_type(y, jnp.bfloat16), jnp.int16), jnp.int32)
    return lax.bitwise_or(lax.bitwise_and(xi, jnp.int32(0xFFFF)),
                          lax.shift_left(yi, 16))


def _mlp_body(s_ref, w1_ref, b1_ref, w2_ref, b2_ref,
              vx_ref, vy_ref, vz_ref, o_ref):
    x = s_ref[...]
    h = jnp.dot(x, w1_ref[...], preferred_element_type=jnp.float32) + b1_ref[...]
    h = h * jax.nn.sigmoid(h)
    nf = jnp.dot(h, w2_ref[...],
                 preferred_element_type=jnp.float32) + b2_ref[...]
    D = x.shape[1]
    o_ref[...] = jnp.concatenate([
        _pack_pair(nf[:, :D], vz_ref[...]),
        _pack_pair(nf[:, D:2 * D], nf[:, 2 * D:]),
        _pack_pair(vx_ref[...], vy_ref[...]),
    ], axis=1)


def _lo16(x):
    return lax.bitcast_convert_type(lax.shift_left(x, 16), jnp.float32)


def _hi16(x):
    return lax.bitcast_convert_type(
        lax.bitwise_and(x, jnp.int32(-65536)), jnp.float32)


def _payload_body(u_ref, d4_ref, g_ref, w4_ref,
                  pa_ref, p0_ref, p1_ref, p2_ref):
    d4 = d4_ref[...]
    uc = d4 * (1.0 / CUTOFF)
    uc2 = uc * uc
    uc3 = uc2 * uc
    uc6 = uc3 * uc3
    uc7 = uc6 * uc
    uc8 = uc7 * uc
    env = 1.0 - 28.0 * uc6 + 48.0 * uc7 - 21.0 * uc8
    env = jnp.where(d4 < CUTOFF, env, 0.0)

    SUB = d4.shape[0]
    nl = lax.broadcasted_iota(jnp.int32, (SUB, 128), 1)
    nb = (nl % 32 + 1).astype(jnp.float32)
    sin4 = jnp.sin(nb * (jnp.pi / CUTOFF) * d4)
    besf = sin4 * (jnp.sqrt(2.0 / CUTOFF) / d4 * env)
    besf = jnp.where(nl % 32 == 31, env, besf)
    filt4 = jnp.dot(besf, w4_ref[...],
                    preferred_element_type=jnp.float32)

    TD = filt4.shape[1] // 4
    D = TD // 3
    for e in range(4):
        rs = pl.ds(e * SUB, SUB)
        base = e * TD
        a = _lo16(g_ref[rs, :D])
        vz = _hi16(g_ref[rs, :D])
        b = _lo16(g_ref[rs, D:2 * D])
        c = _hi16(g_ref[rs, D:2 * D])
        vx = _lo16(g_ref[rs, 2 * D:])
        vy = _hi16(g_ref[rs, 2 * D:])
        bb = b * filt4[:, base + D:base + 2 * D]
        cc = c * filt4[:, base + 2 * D:base + 3 * D]
        pa_ref[rs, :] = a * filt4[:, base:base + D]
        p0_ref[rs, :] = bb * u_ref[rs, 0:1] + cc * vx
        p1_ref[rs, :] = bb * u_ref[rs, 1:2] + cc * vy
        p2_ref[rs, :] = bb * u_ref[rs, 2:3] + cc * vz


def _mlp_pack(scalar_embeddings, w1, b1, w2, b2, vx, vy, vz):
    N, D = scalar_embeddings.shape
    TD = w2.shape[1]
    BN = 2000
    grid = (N // BN,)
    return pl.pallas_call(
        _mlp_body,
        grid=grid,
        in_specs=[
            pl.BlockSpec((BN, D), lambda i: (i, 0)),
            pl.BlockSpec((D, D), lambda i: (0, 0)),
            pl.BlockSpec((1, D), lambda i: (0, 0)),
            pl.BlockSpec((D, TD), lambda i: (0, 0)),
            pl.BlockSpec((1, TD), lambda i: (0, 0)),
            pl.BlockSpec((BN, D), lambda i: (i, 0)),
            pl.BlockSpec((BN, D), lambda i: (i, 0)),
            pl.BlockSpec((BN, D), lambda i: (i, 0)),
        ],
        out_specs=pl.BlockSpec((BN, TD), lambda i: (i, 0)),
        out_shape=jax.ShapeDtypeStruct((N, TD), jnp.int32),
    )(scalar_embeddings, w1, b1.reshape(1, D), w2, b2.reshape(1, TD),
      vx, vy, vz)


BE = 1600


def _payloads(u3, d4, g_all, w4):
    E, TD = g_all.shape
    D = TD // 3
    grid = (E // BE,)
    return pl.pallas_call(
        _payload_body,
        grid=grid,
        in_specs=[
            pl.BlockSpec((BE, 3), lambda i: (i, 0)),
            pl.BlockSpec((BE // 4, 128), lambda i: (i, 0)),
            pl.BlockSpec((BE, TD), lambda i: (i, 0)),
            pl.BlockSpec((128, 4 * TD), lambda i: (0, 0)),
        ],
        out_specs=[pl.BlockSpec((BE, D), lambda i: (i, 0))] * 4,
        out_shape=[jax.ShapeDtypeStruct((E, D), jnp.float32)] * 4,
    )(u3, d4, g_all, w4)


BLK = 128


def _sc_gather(t_all, neigh_pad):
    N, TD = t_all.shape
    E = neigh_pad.shape[0] - BLK
    nblk = E // BLK
    NW = 32
    base_cnt = nblk // NW
    n_extra = nblk - NW * base_cnt
    cnt_max = base_cnt + (1 if n_extra else 0)
    pairs = (cnt_max + 1) // 2
    mesh = plsc.VectorSubcoreMesh(core_axis_name="c", subcore_axis_name="s")

    @functools.partial(
        pl.kernel,
        out_type=jax.ShapeDtypeStruct((E, TD), jnp.int32),
        mesh=mesh,
        scratch_types=[
            pltpu.VMEM((cnt_max * BLK,), jnp.int32),
            pltpu.VMEM((BLK, TD), jnp.int32),
            pltpu.VMEM((BLK, TD), jnp.int32),
            pltpu.SemaphoreType.DMA,
            pltpu.SemaphoreType.DMA,
            pltpu.SemaphoreType.DMA,
            pltpu.SemaphoreType.DMA,
        ],
    )
    def k(t_hbm, idx_hbm, g_hbm, idx_all, buf0, buf1, gs0, gs1, ws0, ws1):
        wid = lax.axis_index("s") * 2 + lax.axis_index("c")
        base_blk = wid * base_cnt + jnp.minimum(wid, n_extra)
        cnt = jnp.where(wid < n_extra, base_cnt + 1, base_cnt)
        ebase = base_blk * BLK

        pltpu.sync_copy(idx_hbm.at[pl.ds(ebase, cnt_max * BLK)], idx_all)

        bufs = (buf0, buf1)
        gsems = (gs0, gs1)
        wsems = (ws0, ws1)

        def gissue(i, b):
            pltpu.async_copy(
                t_hbm.at[idx_all.at[pl.ds(i * BLK, BLK)]], bufs[b], gsems[b])

        def gwait(b):
            pltpu.make_async_copy(
                t_hbm.at[pl.ds(0, BLK)], bufs[b], gsems[b]).wait()

        def wissue(i, b):
            pltpu.async_copy(
                bufs[b], g_hbm.at[pl.ds(ebase + i * BLK, BLK)], wsems[b])

        def wwait(b):
            pltpu.make_async_copy(
                bufs[b], g_hbm.at[pl.ds(0, BLK)], wsems[b]).wait()

        gissue(0, 0)

        @pl.loop(0, pairs)
        def _(ii):
            for b in range(2):
                i = ii * 2 + b

                @pl.when(i < cnt)
                def _(i=i, b=b):
                    gwait(b)
                    wissue(i, b)

                    @pl.when(i + 1 < cnt)
                    def _():
                        @pl.when(i >= 1)
                        def _():
                            wwait(1 - b)

                        gissue(i + 1, 1 - b)

        wwait(0)
        wwait(1)

    return k(t_all, neigh_pad)


def _sc_scatter(c2d_1, c2d_2, pa1, pa2, p01, p02, p11, p12, p21, p22,
                zeros_nd):
    EH, D = pa1.shape
    N = zeros_nd.shape[0]
    nblk = EH // BLK
    NS = 16
    NSH = 8
    CH = ((nblk + NSH - 1) // NSH + 7) // 8 * 8
    n_full = nblk // CH
    rem = nblk - n_full * CH
    cnt_max = CH
    NBUF = 2
    rounds = (cnt_max + NBUF - 1) // NBUF
    tiles = N // 8
    rows8 = (tiles // NS) * 8
    n_hi = tiles - NS * (tiles // NS)
    assert n_hi * (rows8 + 8) + (NS - n_hi) * rows8 == N
    mesh = plsc.VectorSubcoreMesh(core_axis_name="c", subcore_axis_name="s")

    @functools.partial(
        pl.kernel,
        out_type=(jax.ShapeDtypeStruct((N, D), jnp.float32),
                  jax.ShapeDtypeStruct((N, D), jnp.float32),
                  jax.ShapeDtypeStruct((N, D), jnp.float32),
                  jax.ShapeDtypeStruct((N, D), jnp.float32)),
        mesh=mesh,
        scratch_types=[
            pltpu.VMEM((cnt_max, BLK), jnp.int32),
            pltpu.VMEM((BLK, D), jnp.float32),
            pltpu.VMEM((BLK, D), jnp.float32),
            pltpu.VMEM_SHARED((N, D), jnp.float32),
            pltpu.SemaphoreType.DMA,
            pltpu.SemaphoreType.DMA,
            pltpu.SemaphoreType.DMA,
            pltpu.SemaphoreType.DMA,
        ],
    )
    def k(c1_hbm, c2_hbm, pa1_hbm, pa2_hbm, p01_hbm, p02_hbm,
          p11_hbm, p12_hbm, p21_hbm, p22_hbm, z_hbm,
          ds_hbm, v0_hbm, v1_hbm, v2_hbm,
          idx2, b0, b1,
          acc, ps0, ps1, ss0, ss1):
        cid = lax.axis_index("c")
        sid = lax.axis_index("s")
        sidh = lax.rem(sid, NSH)
        sbase = sidh * CH
        cnt = jnp.where(sidh < n_full, CH, rem)
        bufs = (b0, b1)
        psems = (ps0, ps1)
        ssems = (ss0, ss1)

        @pl.when(sid < NSH)
        def _():
            pltpu.sync_copy(c1_hbm.at[pl.ds(sbase, CH)], idx2)

        @pl.when(sid >= NSH)
        def _():
            pltpu.sync_copy(c2_hbm.at[pl.ds(sbase, CH)], idx2)

        def pipe(pay_hbm):
            def pissue(i, b):
                pltpu.async_copy(
                    pay_hbm.at[pl.ds((sbase + i) * BLK, BLK)],
                    bufs[b], psems[b])

            def pwait(b):
                pltpu.make_async_copy(
                    pay_hbm.at[pl.ds(0, BLK)], bufs[b], psems[b]).wait()

            def sissue(i, b):
                pltpu.async_copy(bufs[b], acc.at[idx2.at[i]], ssems[b],
                                 add=True)

            def swait(b):
                pltpu.make_async_copy(
                    pay_hbm.at[pl.ds(0, BLK)], bufs[b], ssems[b]).wait()

            for b in range(NBUF - 1):
                @pl.when(b < cnt)
                def _(b=b):
                    pissue(b, b)

            @pl.loop(0, rounds)
            def _(ii):
                for b in range(NBUF):
                    i = ii * NBUF + b

                    @pl.when(i < cnt)
                    def _(i=i, b=b):
                        pwait(b)
                        sissue(i, b)
                        j = i + NBUF - 1

                        @pl.when(j < cnt)
                        def _():
                            nb = (b + NBUF - 1) % NBUF

                            @pl.when(j >= NBUF)
                            def _():
                                swait(nb)

                            pissue(j, nb)

            for b in range(NBUF):
                swait(b)

        def chunk(payh1_hbm, payh2_hbm, out_hbm):
            @pl.when(sid < n_hi)
            def _():
                start = sid * (rows8 + 8)
                pltpu.sync_copy(z_hbm.at[pl.ds(start, rows8 + 8)],
                                acc.at[pl.ds(start, rows8 + 8)])

            @pl.when(sid >= n_hi)
            def _():
                start = n_hi * (rows8 + 8) + (sid - n_hi) * rows8
                pltpu.sync_copy(z_hbm.at[pl.ds(start, rows8)],
                                acc.at[pl.ds(start, rows8)])

            plsc.subcore_barrier()

            @pl.when(sid < NSH)
            def _():
                pipe(payh1_hbm)

            @pl.when(sid >= NSH)
            def _():
                pipe(payh2_hbm)

            plsc.subcore_barrier()

            @pl.when(sid < n_hi)
            def _():
                start = sid * (rows8 + 8)
                pltpu.sync_copy(acc.at[pl.ds(start, rows8 + 8)],
                                out_hbm.at[pl.ds(start, rows8 + 8)])

            @pl.when(sid >= n_hi)
            def _():
                start = n_hi * (rows8 + 8) + (sid - n_hi) * rows8
                pltpu.sync_copy(acc.at[pl.ds(start, rows8)],
                                out_hbm.at[pl.ds(start, rows8)])

            plsc.subcore_barrier()

        @pl.when(cid == 0)
        def _():
            chunk(pa1_hbm, pa2_hbm, ds_hbm)
            chunk(p01_hbm, p02_hbm, v0_hbm)

        @pl.when(cid == 1)
        def _():
            chunk(p11_hbm, p12_hbm, v1_hbm)
            chunk(p21_hbm, p22_hbm, v2_hbm)

    return k(c2d_1, c2d_2, pa1, pa2, p01, p02, p11, p12, p21, p22, zeros_nd)


def kernel(vector_embeddings, scalar_embeddings, edge_vectors, edge_index,
           Wf, bf, w1, b1, w2, b2):
    N, D = scalar_embeddings.shape
    TD = 3 * D
    E = edge_vectors.shape[0]
    central = edge_index[0].astype(jnp.int32)
    neigh = edge_index[1].astype(jnp.int32)

    t_all = _mlp_pack(scalar_embeddings, w1, b1, w2, b2,
                      vector_embeddings[:, :, 0],
                      vector_embeddings[:, :, 1],
                      vector_embeddings[:, :, 2])

    d = jnp.sqrt(jnp.sum(edge_vectors * edge_vectors, axis=-1, keepdims=True))
    u3 = edge_vectors / d
    d4 = d.reshape(E // BE, 4, BE // 4).transpose(0, 2, 1).reshape(E // 4, 4)
    d4 = jnp.repeat(d4, 32, axis=1)
    wf32 = jnp.zeros((32, TD), jnp.float32).at[:NBASIS].set(Wf).at[31].set(bf)
    w4 = jax.scipy.linalg.block_diag(wf32, wf32, wf32, wf32)

    EH = E // 2
    pays = []
    for h in range(2):
        sl = slice(h * EH, (h + 1) * EH)
        neigh_pad = jnp.concatenate([neigh[sl], jnp.zeros((BLK,), jnp.int32)])
        g_h = _sc_gather(t_all, neigh_pad)
        pays.append(_payloads(u3[sl], d4[h * (EH // 4):(h + 1) * (EH // 4)],
                              g_h, w4))

    zeros_nd = jnp.zeros((N, D), jnp.float32)
    nblk_h = EH // BLK
    ch = ((nblk_h + 7) // 8 + 7) // 8 * 8
    c2d = []
    for h in range(2):
        c_h = central[h * EH:(h + 1) * EH].reshape(nblk_h, BLK)
        c2d.append(jnp.concatenate(
            [c_h, jnp.zeros((8 * ch - nblk_h, BLK), jnp.int32)]))
    (pa1, p01, p11, p21), (pa2, p02, p12, p22) = pays
    ds, v0, v1, v2 = _sc_scatter(c2d[0], c2d[1], pa1, pa2, p01, p02,
                                 p11, p12, p21, p22, zeros_nd)

    delta_v = jnp.stack([v0, v1, v2], axis=-1)
    return (delta_v, ds)


# speedup vs baseline: 1.7618x; 1.1161x over previous
import functools

import jax
import jax.numpy as jnp
from jax import lax
from jax.experimental import pallas as pl
from jax.experimental.pallas import tpu as pltpu
from jax.experimental.pallas import tpu_sc as plsc

CUTOFF = 5.0
NBASIS = 20


def _pack_pair(x, y):
    xi = lax.convert_element_type(
        lax.bitcast_convert_type(
            lax.convert_element_type(x, jnp.bfloat16), jnp.int16), jnp.int32)
    yi = lax.convert_element_type(
        lax.bitcast_convert_type(
            lax.convert_element_type(y, jnp.bfloat16), jnp.int16), jnp.int32)
    return lax.bitwise_or(lax.bitwise_and(xi, jnp.int32(0xFFFF)),
                          lax.shift_left(yi, 16))


def _mlp_body(s_ref, w1_ref, b1_ref, w2_ref, b2_ref,
              vx_ref, vy_ref, vz_ref, o_ref):
    x = s_ref[...]
    h = jnp.dot(x, w1_ref[...], preferred_element_type=jnp.float32) + b1_ref[...]
    h = h * jax.nn.sigmoid(h)
    nf = jnp.dot(h, w2_ref[...],
                 preferred_element_type=jnp.float32) + b2_ref[...]
    D = x.shape[1]
    o_ref[...] = jnp.concatenate([
        _pack_pair(nf[:, :D], vz_ref[...]),
        _pack_pair(nf[:, D:2 * D], nf[:, 2 * D:]),
        _pack_pair(vx_ref[...], vy_ref[...]),
    ], axis=1)


def _lo16(x):
    return lax.bitcast_convert_type(lax.shift_left(x, 16), jnp.float32)


def _hi16(x):
    return lax.bitcast_convert_type(
        lax.bitwise_and(x, jnp.int32(-65536)), jnp.float32)


def _payload_body(ev_ref, d4_ref, g_ref, w4_ref,
                  pa_ref, p0_ref, p1_ref, p2_ref):
    SUB = g_ref.shape[0] // 4
    d4 = d4_ref[...]
    us = []
    for e in range(4):
        rs = pl.ds(e * SUB, SUB)
        inv = 1.0 / d4[:, 32 * e:32 * e + 1]
        us.append((ev_ref[rs, 0:1] * inv,
                   ev_ref[rs, 1:2] * inv,
                   ev_ref[rs, 2:3] * inv))
    uc = d4 * (1.0 / CUTOFF)
    uc2 = uc * uc
    uc3 = uc2 * uc
    uc6 = uc3 * uc3
    uc7 = uc6 * uc
    uc8 = uc7 * uc
    env = 1.0 - 28.0 * uc6 + 48.0 * uc7 - 21.0 * uc8
    env = jnp.where(d4 < CUTOFF, env, 0.0)

    nl = lax.broadcasted_iota(jnp.int32, (SUB, 128), 1)
    nb = (nl % 32 + 1).astype(jnp.float32)
    sin4 = jnp.sin(nb * (jnp.pi / CUTOFF) * d4)
    besf = sin4 * (jnp.sqrt(2.0 / CUTOFF) / d4 * env)
    besf = jnp.where(nl % 32 == 31, env, besf)
    filt4 = jnp.dot(besf, w4_ref[...],
                    preferred_element_type=jnp.float32)

    TD = filt4.shape[1] // 4
    D = TD // 3
    for e in range(4):
        rs = pl.ds(e * SUB, SUB)
        base = e * TD
        a = _lo16(g_ref[rs, :D])
        vz = _hi16(g_ref[rs, :D])
        b = _lo16(g_ref[rs, D:2 * D])
        c = _hi16(g_ref[rs, D:2 * D])
        vx = _lo16(g_ref[rs, 2 * D:])
        vy = _hi16(g_ref[rs, 2 * D:])
        bb = b * filt4[:, base + D:base + 2 * D]
        cc = c * filt4[:, base + 2 * D:base + 3 * D]
        pa_ref[rs, :] = a * filt4[:, base:base + D]
        p0_ref[rs, :] = bb * us[e][0] + cc * vx
        p1_ref[rs, :] = bb * us[e][1] + cc * vy
        p2_ref[rs, :] = bb * us[e][2] + cc * vz


def _mlp_pack(scalar_embeddings, w1, b1, w2, b2, vx, vy, vz):
    N, D = scalar_embeddings.shape
    TD = w2.shape[1]
    BN = 2000
    grid = (N // BN,)
    return pl.pallas_call(
        _mlp_body,
        grid=grid,
        in_specs=[
            pl.BlockSpec((BN, D), lambda i: (i, 0)),
            pl.BlockSpec((D, D), lambda i: (0, 0)),
            pl.BlockSpec((1, D), lambda i: (0, 0)),
            pl.BlockSpec((D, TD), lambda i: (0, 0)),
            pl.BlockSpec((1, TD), lambda i: (0, 0)),
            pl.BlockSpec((BN, D), lambda i: (i, 0)),
            pl.BlockSpec((BN, D), lambda i: (i, 0)),
            pl.BlockSpec((BN, D), lambda i: (i, 0)),
        ],
        out_specs=pl.BlockSpec((BN, TD), lambda i: (i, 0)),
        out_shape=jax.ShapeDtypeStruct((N, TD), jnp.int32),
    )(scalar_embeddings, w1, b1.reshape(1, D), w2, b2.reshape(1, TD),
      vx, vy, vz)


BE = 1600


def _payloads(ev, d4, g_all, w4):
    E, TD = g_all.shape
    D = TD // 3
    grid = (E // BE,)
    return pl.pallas_call(
        _payload_body,
        grid=grid,
        in_specs=[
            pl.BlockSpec((BE, 3), lambda i: (i, 0)),
            pl.BlockSpec((BE // 4, 128), lambda i: (i, 0)),
            pl.BlockSpec((BE, TD), lambda i: (i, 0)),
            pl.BlockSpec((128, 4 * TD), lambda i: (0, 0)),
        ],
        out_specs=[pl.BlockSpec((BE, D), lambda i: (i, 0))] * 4,
        out_shape=[jax.ShapeDtypeStruct((E, D), jnp.float32)] * 4,
    )(ev, d4, g_all, w4)


BLK = 128


def _sc_gather(t_all, neigh_pad):
    N, TD = t_all.shape
    E = neigh_pad.shape[0] - BLK
    nblk = E // BLK
    NW = 32
    base_cnt = nblk // NW
    n_extra = nblk - NW * base_cnt
    cnt_max = base_cnt + (1 if n_extra else 0)
    pairs = (cnt_max + 1) // 2
    mesh = plsc.VectorSubcoreMesh(core_axis_name="c", subcore_axis_name="s")

    @functools.partial(
        pl.kernel,
        out_type=jax.ShapeDtypeStruct((E, TD), jnp.int32),
        mesh=mesh,
        scratch_types=[
            pltpu.VMEM((cnt_max * BLK,), jnp.int32),
            pltpu.VMEM((BLK, TD), jnp.int32),
            pltpu.VMEM((BLK, TD), jnp.int32),
            pltpu.SemaphoreType.DMA,
            pltpu.SemaphoreType.DMA,
            pltpu.SemaphoreType.DMA,
            pltpu.SemaphoreType.DMA,
        ],
    )
    def k(t_hbm, idx_hbm, g_hbm, idx_all, buf0, buf1, gs0, gs1, ws0, ws1):
        wid = lax.axis_index("s") * 2 + lax.axis_index("c")
        base_blk = wid * base_cnt + jnp.minimum(wid, n_extra)
        cnt = jnp.where(wid < n_extra, base_cnt + 1, base_cnt)
        ebase = base_blk * BLK

        pltpu.sync_copy(idx_hbm.at[pl.ds(ebase, cnt_max * BLK)], idx_all)

        bufs = (buf0, buf1)
        gsems = (gs0, gs1)
        wsems = (ws0, ws1)

        def gissue(i, b):
            pltpu.async_copy(
                t_hbm.at[idx_all.at[pl.ds(i * BLK, BLK)]], bufs[b], gsems[b])

        def gwait(b):
            pltpu.make_async_copy(
                t_hbm.at[pl.ds(0, BLK)], bufs[b], gsems[b]).wait()

        def wissue(i, b):
            pltpu.async_copy(
                bufs[b], g_hbm.at[pl.ds(ebase + i * BLK, BLK)], wsems[b])

        def wwait(b):
            pltpu.make_async_copy(
                bufs[b], g_hbm.at[pl.ds(0, BLK)], wsems[b]).wait()

        gissue(0, 0)

        @pl.loop(0, pairs)
        def _(ii):
            for b in range(2):
                i = ii * 2 + b

                @pl.when(i < cnt)
                def _(i=i, b=b):
                    gwait(b)
                    wissue(i, b)

                    @pl.when(i + 1 < cnt)
                    def _():
                        @pl.when(i >= 1)
                        def _():
                            wwait(1 - b)

                        gissue(i + 1, 1 - b)

        wwait(0)
        wwait(1)

    return k(t_all, neigh_pad)


def _sc_scatter(c2d_1, c2d_2, pa1, pa2, p01, p02, p11, p12, p21, p22,
                zeros_nd):
    EH, D = pa1.shape
    N = zeros_nd.shape[0]
    nblk = EH // BLK
    NS = 16
    NSH = 8
    CH = ((nblk + NSH - 1) // NSH + 7) // 8 * 8
    n_full = nblk // CH
    rem = nblk - n_full * CH
    cnt_max = CH
    NBUF = 2
    rounds = (cnt_max + NBUF - 1) // NBUF
    tiles = N // 8
    rows8 = (tiles // NS) * 8
    n_hi = tiles - NS * (tiles // NS)
    assert n_hi * (rows8 + 8) + (NS - n_hi) * rows8 == N
    mesh = plsc.VectorSubcoreMesh(core_axis_name="c", subcore_axis_name="s")

    @functools.partial(
        pl.kernel,
        out_type=(jax.ShapeDtypeStruct((N, D), jnp.float32),
                  jax.ShapeDtypeStruct((N, D), jnp.float32),
                  jax.ShapeDtypeStruct((N, D), jnp.float32),
                  jax.ShapeDtypeStruct((N, D), jnp.float32)),
        mesh=mesh,
        scratch_types=[
            pltpu.VMEM((cnt_max, BLK), jnp.int32),
            pltpu.VMEM((BLK, D), jnp.float32),
            pltpu.VMEM((BLK, D), jnp.float32),
            pltpu.VMEM_SHARED((N, D), jnp.float32),
            pltpu.SemaphoreType.DMA,
            pltpu.SemaphoreType.DMA,
            pltpu.SemaphoreType.DMA,
            pltpu.SemaphoreType.DMA,
        ],
    )
    def k(c1_hbm, c2_hbm, pa1_hbm, pa2_hbm, p01_hbm, p02_hbm,
          p11_hbm, p12_hbm, p21_hbm, p22_hbm, z_hbm,
          ds_hbm, v0_hbm, v1_hbm, v2_hbm,
          idx2, b0, b1,
          acc, ps0, ps1, ss0, ss1):
        cid = lax.axis_index("c")
        sid = lax.axis_index("s")
        sidh = lax.rem(sid, NSH)
        sbase = sidh * CH
        cnt = jnp.where(sidh < n_full, CH, rem)
        bufs = (b0, b1)
        psems = (ps0, ps1)
        ssems = (ss0, ss1)

        @pl.when(sid < NSH)
        def _():
            pltpu.sync_copy(c1_hbm.at[pl.ds(sbase, CH)], idx2)

        @pl.when(sid >= NSH)
        def _():
            pltpu.sync_copy(c2_hbm.at[pl.ds(sbase, CH)], idx2)

        def pipe(pay_hbm):
            def pissue(i, b):
                pltpu.async_copy(
                    pay_hbm.at[pl.ds((sbase + i) * BLK, BLK)],
                    bufs[b], psems[b])

            def pwait(b):
                pltpu.make_async_copy(
                    pay_hbm.at[pl.ds(0, BLK)], bufs[b], psems[b]).wait()

            def sissue(i, b):
                pltpu.async_copy(bufs[b], acc.at[idx2.at[i]], ssems[b],
                                 add=True)

            def swait(b):
                pltpu.make_async_copy(
                    pay_hbm.at[pl.ds(0, BLK)], bufs[b], ssems[b]).wait()

            for b in range(NBUF - 1):
                @pl.when(b < cnt)
                def _(b=b):
                    pissue(b, b)

            @pl.loop(0, rounds)
            def _(ii):
                for b in range(NBUF):
                    i = ii * NBUF + b

                    @pl.when(i < cnt)
                    def _(i=i, b=b):
                        pwait(b)
                        sissue(i, b)
                        j = i + NBUF - 1

                        @pl.when(j < cnt)
                        def _():
                            nb = (b + NBUF - 1) % NBUF

                            @pl.when(j >= NBUF)
                            def _():
                                swait(nb)

                            pissue(j, nb)

            for b in range(NBUF):
                swait(b)

        def chunk(payh1_hbm, payh2_hbm, out_hbm):
            @pl.when(sid < n_hi)
            def _():
                start = sid * (rows8 + 8)
                pltpu.sync_copy(z_hbm.at[pl.ds(start, rows8 + 8)],
                                acc.at[pl.ds(start, rows8 + 8)])

            @pl.when(sid >= n_hi)
            def _():
                start = n_hi * (rows8 + 8) + (sid - n_hi) * rows8
                pltpu.sync_copy(z_hbm.at[pl.ds(start, rows8)],
                                acc.at[pl.ds(start, rows8)])

            plsc.subcore_barrier()

            @pl.when(sid < NSH)
            def _():
                pipe(payh1_hbm)

            @pl.when(sid >= NSH)
            def _():
                pipe(payh2_hbm)

            plsc.subcore_barrier()

            @pl.when(sid < n_hi)
            def _():
                start = sid * (rows8 + 8)
                pltpu.sync_copy(acc.at[pl.ds(start, rows8 + 8)],
                                out_hbm.at[pl.ds(start, rows8 + 8)])

            @pl.when(sid >= n_hi)
            def _():
                start = n_hi * (rows8 + 8) + (sid - n_hi) * rows8
                pltpu.sync_copy(acc.at[pl.ds(start, rows8)],
                                out_hbm.at[pl.ds(start, rows8)])

            plsc.subcore_barrier()

        @pl.when(cid == 0)
        def _():
            chunk(pa1_hbm, pa2_hbm, ds_hbm)
            chunk(p01_hbm, p02_hbm, v0_hbm)

        @pl.when(cid == 1)
        def _():
            chunk(p11_hbm, p12_hbm, v1_hbm)
            chunk(p21_hbm, p22_hbm, v2_hbm)

    return k(c2d_1, c2d_2, pa1, pa2, p01, p02, p11, p12, p21, p22, zeros_nd)


def kernel(vector_embeddings, scalar_embeddings, edge_vectors, edge_index,
           Wf, bf, w1, b1, w2, b2):
    N, D = scalar_embeddings.shape
    TD = 3 * D
    E = edge_vectors.shape[0]
    central = edge_index[0].astype(jnp.int32)
    neigh = edge_index[1].astype(jnp.int32)

    t_all = _mlp_pack(scalar_embeddings, w1, b1, w2, b2,
                      vector_embeddings[:, :, 0],
                      vector_embeddings[:, :, 1],
                      vector_embeddings[:, :, 2])

    d = jnp.sqrt(jnp.sum(edge_vectors * edge_vectors, axis=-1, keepdims=True))
    d4 = d.reshape(E // BE, 4, BE // 4).transpose(0, 2, 1).reshape(E // 4, 4)
    d4 = jnp.repeat(d4, 32, axis=1)
    wf32 = jnp.zeros((32, TD), jnp.float32).at[:NBASIS].set(Wf).at[31].set(bf)
    w4 = jax.scipy.linalg.block_diag(wf32, wf32, wf32, wf32)

    EH = E // 2
    pays = []
    for h in range(2):
        sl = slice(h * EH, (h + 1) * EH)
        neigh_pad = jnp.concatenate([neigh[sl], jnp.zeros((BLK,), jnp.int32)])
        g_h = _sc_gather(t_all, neigh_pad)
        pays.append(_payloads(edge_vectors[sl],
                              d4[h * (EH // 4):(h + 1) * (EH // 4)], g_h, w4))

    zeros_nd = jnp.zeros((N, D), jnp.float32)
    nblk_h = EH // BLK
    ch = ((nblk_h + 7) // 8 + 7) // 8 * 8
    c2d = []
    for h in range(2):
        c_h = central[h * EH:(h + 1) * EH].reshape(nblk_h, BLK)
        c2d.append(jnp.concatenate(
            [c_h, jnp.zeros((8 * ch - nblk_h, BLK), jnp.int32)]))
    (pa1, p01, p11, p21), (pa2, p02, p12, p22) = pays
    ds, v0, v1, v2 = _sc_scatter(c2d[0], c2d[1], pa1, pa2, p01, p02,
                                 p11, p12, p21, p22, zeros_nd)

    delta_v = jnp.stack([v0, v1, v2], axis=-1)
    return (delta_v, ds)
